# baseline (speedup 1.0000x reference)
import jax
import jax.numpy as jnp
from jax.experimental import pallas as pl
from jax.experimental.pallas import tpu as pltpu

_NV = 8
_NSTEP = 5
_NSLAB = 37


def _lrelu(x, alpha):
    return jnp.where(x >= 0, x, alpha * x)


def _gat_conv_dense(x, eaT, src, dst, W, a_s, a_d, We, a_e, b):
    E = src.shape[1]
    N = x.shape[0]
    f32 = jnp.float32
    lhsT = (((0,), (0,)), ((), ()))
    h = jnp.dot(x, W, preferred_element_type=f32)
    hs = jnp.sum(h * a_s, axis=1, keepdims=True)
    hd = jnp.sum(h * a_d, axis=1, keepdims=True)
    wae = jnp.sum(We * a_e, axis=1, keepdims=True)
    el = jax.lax.dot_general(wae, eaT, lhsT,
                             preferred_element_type=f32)

    iota = jax.lax.broadcasted_iota(jnp.int32, (N, E), 0)
    ohs = (iota == src).astype(f32)
    mask = iota == dst
    ohd = mask.astype(f32)

    lg = _lrelu(jax.lax.dot_general(hs, ohs, lhsT, preferred_element_type=f32)
                + jax.lax.dot_general(hd, ohd, lhsT, preferred_element_type=f32)
                + el, 0.2)

    neg_inf = jnp.float32(-jnp.inf)
    m = jnp.max(jnp.where(mask, lg, neg_inf), axis=1, keepdims=True)
    ex = jnp.exp(jnp.where(mask, lg - m, neg_inf))
    s = jnp.sum(ex, axis=1, keepdims=True)
    alpha = ex / (s + 1e-16)

    hsrc = jax.lax.dot_general(ohs, h, lhsT,
                               preferred_element_type=f32)
    return jnp.dot(alpha, hsrc, preferred_element_type=f32) + b


def _mega_kernel(*refs):
    (emg_x, emg_eaT, emg_ei,
     eeg_x, eeg_eaT, eeg_ei,
     eW1, eas1, ead1, eWe1, eae1, eb1, eW2, eas2, ead2, eWe2, eae2, eb2,
     epW, epb,
     gW1, gas1, gad1, gWe1, gae1, gb1, gW2, gas2, gad2, gWe2, gae2, gb2,
     gpW, gpb,
     eps1, eps2, g1W1, g1b1, g1W2, g1b2, g2W1, g2b1, g2W2, g2b2,
     w_full,
     d1b, d2wa, d2wb, d2b2d, d3wa, d3wb, d3b, d4w, d4b, d5w, d5b,
     out_r, ze, zg, acc) = refs
    j = pl.program_id(0)
    f32 = jnp.float32
    rr = lambda r: r[...].reshape(1, -1)

    @pl.when(j == 0)
    def _():
        def branch(x_r, eaT_r, ei_r, W1, as1, ad1, We1, ae1, b1,
                   W2, as2, ad2, We2, ae2, b2, pW, pb):
            ei = ei_r[...]
            src = ei[0:1, :]
            dst = ei[1:2, :]
            h1 = jax.nn.relu(_gat_conv_dense(
                x_r[...], eaT_r[...], src, dst, W1[...], rr(as1), rr(ad1),
                We1[...], rr(ae1), rr(b1)))
            h2 = _gat_conv_dense(
                h1, eaT_r[...], src, dst, W2[...], rr(as2), rr(ad2),
                We2[...], rr(ae2), rr(b2))
            return jnp.dot(h2, pW[...], preferred_element_type=f32) + rr(pb)

        f_emg = branch(emg_x, emg_eaT, emg_ei,
                       eW1, eas1, ead1, eWe1, eae1, eb1,
                       eW2, eas2, ead2, eWe2, eae2, eb2, epW, epb)
        f_eeg = branch(eeg_x, eeg_eaT, eeg_ei,
                       gW1, gas1, gad1, gWe1, gae1, gb1,
                       gW2, gas2, gad2, gWe2, gae2, gb2, gpW, gpb)

        def gin(a, bpart, eps, W1, b1, W2, b2):
            tot = (jnp.sum(a, axis=0, keepdims=True)
                   + jnp.sum(bpart, axis=0, keepdims=True))
            scale = 1.0 + eps[0, 0]

            def one(t):
                hh = scale * t + tot
                return jnp.dot(jax.nn.relu(
                    jnp.dot(hh, W1[...], preferred_element_type=f32)
                    + rr(b1)), W2[...], preferred_element_type=f32) + rr(b2)

            return one(a), one(bpart)

        h1e, h1g = gin(f_emg, f_eeg, eps1, g1W1, g1b1, g1W2, g1b2)
        h1e, h1g = jax.nn.relu(h1e), jax.nn.relu(h1g)
        h2e, h2g = gin(h1e, h1g, eps2, g2W1, g2b1, g2W2, g2b2)
        ze[0:21, :] = h2e
        zg[...] = h2g
        acc[...] = rr(d1b)

    for v in range(_NV):
        idx = _NV * j + v

        @pl.when(idx < _NSLAB)
        def _(idx=idx):
            ie = jnp.minimum(idx, 20)
            ig = jnp.clip(idx - 21, 0, 15)
            zrow = jnp.where(idx < 21, ze[pl.ds(ie, 1), :],
                             zg[pl.ds(ig, 1), :])
            slab = w_full[pl.ds(idx * 128, 128), :]
            acc[...] += jnp.dot(zrow, slab, preferred_element_type=f32)

    @pl.when(j == _NSTEP - 1)
    def _():
        t1 = _lrelu(acc[...], 0.01)
        d2b = rr(d2b2d)
        ta = _lrelu(jnp.dot(t1, d2wa[...], preferred_element_type=f32)
                    + d2b[:, :512], 0.01)
        tb = _lrelu(jnp.dot(t1, d2wb[...], preferred_element_type=f32)
                    + d2b[:, 512:], 0.01)
        t = _lrelu(jnp.dot(ta, d3wa[...], preferred_element_type=f32)
                   + jnp.dot(tb, d3wb[...], preferred_element_type=f32)
                   + rr(d3b), 0.01)
        t = _lrelu(jnp.dot(t, d4w[...], preferred_element_type=f32)
                   + rr(d4b), 0.01)
        out_r[...] = (jnp.dot(t, d5w[...], preferred_element_type=f32)
                      + rr(d5b))


def kernel(emg_x, emg_edge_index, emg_edge_attr, eeg_x, eeg_edge_index,
           eeg_edge_attr, params):
    p = params
    f32 = jnp.float32

    front_in = [
        emg_x, emg_edge_attr.T, emg_edge_index,
        eeg_x, eeg_edge_attr.T, eeg_edge_index,
        p['emg_W1'], p['emg_as1'], p['emg_ad1'], p['emg_We1'],
        p['emg_ae1'], p['emg_b1'],
        p['emg_W2'], p['emg_as2'], p['emg_ad2'], p['emg_We2'],
        p['emg_ae2'], p['emg_b2'],
        p['emg_proj_W'], p['emg_proj_b'],
        p['eeg_W1'], p['eeg_as1'], p['eeg_ad1'], p['eeg_We1'],
        p['eeg_ae1'], p['eeg_b1'],
        p['eeg_W2'], p['eeg_as2'], p['eeg_ad2'], p['eeg_We2'],
        p['eeg_ae2'], p['eeg_b2'],
        p['eeg_proj_W'], p['eeg_proj_b'],
        p['gin_eps1'].reshape(1, 1), p['gin_eps2'].reshape(1, 1),
        p['g1W1'], p['g1b1'], p['g1W2'], p['g1b2'],
        p['g2W1'], p['g2b1'], p['g2W2'], p['g2b2'],
    ]
    const2 = lambda a: pl.BlockSpec(a.shape, lambda j: (0,) * a.ndim)
    in_specs = [const2(a) for a in front_in]

    mega_in = list(front_in)
    mega_in.append(p['d1W'])
    in_specs.append(pl.BlockSpec((4736, 2056), lambda j: (0, 0)))
    tail_in = [p['d1b'], p['d2W'], p['d2W'], p['d2b'], p['d3W'], p['d3W'],
               p['d3b'], p['d4W'], p['d4b'], p['d5W'], p['d5b']]
    tail_specs = [const2(p['d1b']),
                  pl.BlockSpec((2056, 512), lambda j: (0, 0)),
                  pl.BlockSpec((2056, 512), lambda j: (0, 1)),
                  const2(p['d2b']),
                  pl.BlockSpec((512, 512), lambda j: (0, 0)),
                  pl.BlockSpec((512, 512), lambda j: (1, 0)),
                  const2(p['d3b']), const2(p['d4W']), const2(p['d4b']),
                  const2(p['d5W']), const2(p['d5b'])]
    mega_in += tail_in
    in_specs += tail_specs

    out = pl.pallas_call(
        _mega_kernel,
        grid=(_NSTEP,),
        in_specs=in_specs,
        out_specs=pl.BlockSpec((1, 8), lambda j: (0, 0)),
        out_shape=jax.ShapeDtypeStruct((1, 8), f32),
        scratch_shapes=[pltpu.VMEM((24, 128), f32),
                        pltpu.VMEM((16, 128), f32),
                        pltpu.VMEM((1, 2056), f32)],
        compiler_params=pltpu.CompilerParams(
            dimension_semantics=("arbitrary",),
            vmem_limit_bytes=112 * 1024 * 1024,
        ),
    )(*mega_in)
    return out


# speedup vs baseline: 1.0789x; 1.0789x over previous
import jax
import jax.numpy as jnp
from jax.experimental import pallas as pl
from jax.experimental.pallas import tpu as pltpu

_NV = 8
_NSTEP = 5
_NSLAB = 37


def _lrelu(x, alpha):
    return jnp.where(x >= 0, x, alpha * x)


def _gat_conv_dense(x, eaT, src, dst, W, a_s, a_d, We, a_e, b):
    E = src.shape[1]
    N = x.shape[0]
    f32 = jnp.float32
    lhsT = (((0,), (0,)), ((), ()))
    h = jnp.dot(x, W, preferred_element_type=f32)
    hs = jnp.sum(h * a_s, axis=1, keepdims=True)
    hd = jnp.sum(h * a_d, axis=1, keepdims=True)
    wae = jnp.sum(We * a_e, axis=1, keepdims=True)
    el = jax.lax.dot_general(wae, eaT, lhsT,
                             preferred_element_type=f32)

    iota = jax.lax.broadcasted_iota(jnp.int32, (N, E), 0)
    ohs = (iota == src).astype(f32)
    mask = iota == dst
    ohd = mask.astype(f32)

    lg = _lrelu(jax.lax.dot_general(hs, ohs, lhsT, preferred_element_type=f32)
                + jax.lax.dot_general(hd, ohd, lhsT, preferred_element_type=f32)
                + el, 0.2)

    neg_inf = jnp.float32(-jnp.inf)
    m = jnp.max(jnp.where(mask, lg, neg_inf), axis=1, keepdims=True)
    ex = jnp.exp(jnp.where(mask, lg - m, neg_inf))
    s = jnp.sum(ex, axis=1, keepdims=True)
    alpha = ex / (s + 1e-16)

    hsrc = jax.lax.dot_general(ohs, h, lhsT,
                               preferred_element_type=f32)
    return jnp.dot(alpha, hsrc, preferred_element_type=f32) + b


def _mega_kernel(*refs):
    (emg_x, emg_eaT, emg_ei,
     eeg_x, eeg_eaT, eeg_ei,
     eW1, eas1, ead1, eWe1, eae1, eb1, eW2, eas2, ead2, eWe2, eae2, eb2,
     epW, epb,
     gW1, gas1, gad1, gWe1, gae1, gb1, gW2, gas2, gad2, gWe2, gae2, gb2,
     gpW, gpb,
     eps1, eps2, g1W1, g1b1, g1W2, g1b2, g2W1, g2b1, g2W2, g2b2,
     w0, w1, w2, w3, w4, w5, w6, w7,
     d1b, d2wa, d2wb, d2b2d, d3wa, d3wb, d3b, d4w, d4b, d5w, d5b,
     out_r, ze, zg, acc) = refs
    _W_VIEWS = (w0, w1, w2, w3, w4, w5, w6, w7)
    j = pl.program_id(0)
    f32 = jnp.float32
    rr = lambda r: r[...].reshape(1, -1)

    @pl.when(j == 0)
    def _():
        def branch(x_r, eaT_r, ei_r, W1, as1, ad1, We1, ae1, b1,
                   W2, as2, ad2, We2, ae2, b2, pW, pb):
            ei = ei_r[...]
            src = ei[0:1, :]
            dst = ei[1:2, :]
            h1 = jax.nn.relu(_gat_conv_dense(
                x_r[...], eaT_r[...], src, dst, W1[...], rr(as1), rr(ad1),
                We1[...], rr(ae1), rr(b1)))
            h2 = _gat_conv_dense(
                h1, eaT_r[...], src, dst, W2[...], rr(as2), rr(ad2),
                We2[...], rr(ae2), rr(b2))
            return jnp.dot(h2, pW[...], preferred_element_type=f32) + rr(pb)

        f_emg = branch(emg_x, emg_eaT, emg_ei,
                       eW1, eas1, ead1, eWe1, eae1, eb1,
                       eW2, eas2, ead2, eWe2, eae2, eb2, epW, epb)
        f_eeg = branch(eeg_x, eeg_eaT, eeg_ei,
                       gW1, gas1, gad1, gWe1, gae1, gb1,
                       gW2, gas2, gad2, gWe2, gae2, gb2, gpW, gpb)

        def gin(a, bpart, eps, W1, b1, W2, b2):
            tot = (jnp.sum(a, axis=0, keepdims=True)
                   + jnp.sum(bpart, axis=0, keepdims=True))
            scale = 1.0 + eps[0, 0]

            def one(t):
                hh = scale * t + tot
                return jnp.dot(jax.nn.relu(
                    jnp.dot(hh, W1[...], preferred_element_type=f32)
                    + rr(b1)), W2[...], preferred_element_type=f32) + rr(b2)

            return one(a), one(bpart)

        h1e, h1g = gin(f_emg, f_eeg, eps1, g1W1, g1b1, g1W2, g1b2)
        h1e, h1g = jax.nn.relu(h1e), jax.nn.relu(h1g)
        h2e, h2g = gin(h1e, h1g, eps2, g2W1, g2b1, g2W2, g2b2)
        ze[0:21, :] = h2e
        zg[...] = h2g
        acc[...] = rr(d1b)

    for v, w_r in enumerate(_W_VIEWS):
        idx = _NV * j + v

        @pl.when(idx < _NSLAB)
        def _(idx=idx, w_r=w_r):
            ie = jnp.minimum(idx, 20)
            ig = jnp.clip(idx - 21, 0, 15)
            zrow = jnp.where(idx < 21, ze[pl.ds(ie, 1), :],
                             zg[pl.ds(ig, 1), :])
            acc[...] += jnp.dot(zrow, w_r[...], preferred_element_type=f32)

    @pl.when(j == _NSTEP - 1)
    def _():
        t1 = _lrelu(acc[...], 0.01)
        d2b = rr(d2b2d)
        ta = _lrelu(jnp.dot(t1, d2wa[...], preferred_element_type=f32)
                    + d2b[:, :512], 0.01)
        tb = _lrelu(jnp.dot(t1, d2wb[...], preferred_element_type=f32)
                    + d2b[:, 512:], 0.01)
        t = _lrelu(jnp.dot(ta, d3wa[...], preferred_element_type=f32)
                   + jnp.dot(tb, d3wb[...], preferred_element_type=f32)
                   + rr(d3b), 0.01)
        t = _lrelu(jnp.dot(t, d4w[...], preferred_element_type=f32)
                   + rr(d4b), 0.01)
        out_r[...] = (jnp.dot(t, d5w[...], preferred_element_type=f32)
                      + rr(d5b))


def kernel(emg_x, emg_edge_index, emg_edge_attr, eeg_x, eeg_edge_index,
           eeg_edge_attr, params):
    p = params
    f32 = jnp.float32

    front_in = [
        emg_x, emg_edge_attr.T, emg_edge_index,
        eeg_x, eeg_edge_attr.T, eeg_edge_index,
        p['emg_W1'], p['emg_as1'], p['emg_ad1'], p['emg_We1'],
        p['emg_ae1'], p['emg_b1'],
        p['emg_W2'], p['emg_as2'], p['emg_ad2'], p['emg_We2'],
        p['emg_ae2'], p['emg_b2'],
        p['emg_proj_W'], p['emg_proj_b'],
        p['eeg_W1'], p['eeg_as1'], p['eeg_ad1'], p['eeg_We1'],
        p['eeg_ae1'], p['eeg_b1'],
        p['eeg_W2'], p['eeg_as2'], p['eeg_ad2'], p['eeg_We2'],
        p['eeg_ae2'], p['eeg_b2'],
        p['eeg_proj_W'], p['eeg_proj_b'],
        p['gin_eps1'].reshape(1, 1), p['gin_eps2'].reshape(1, 1),
        p['g1W1'], p['g1b1'], p['g1W2'], p['g1b2'],
        p['g2W1'], p['g2b1'], p['g2W2'], p['g2b2'],
    ]
    const2 = lambda a: pl.BlockSpec(a.shape, lambda j: (0,) * a.ndim)
    in_specs = [const2(a) for a in front_in]

    mega_in = list(front_in)
    for v in range(_NV):
        mega_in.append(p['d1W'])
        in_specs.append(pl.BlockSpec(
            (128, 2056), lambda j, v=v: (jnp.minimum(_NV * j + v, _NSLAB - 1), 0)))
    tail_in = [p['d1b'], p['d2W'], p['d2W'], p['d2b'], p['d3W'], p['d3W'],
               p['d3b'], p['d4W'], p['d4b'], p['d5W'], p['d5b']]
    tail_specs = [const2(p['d1b']),
                  pl.BlockSpec((2056, 512), lambda j: (0, 0)),
                  pl.BlockSpec((2056, 512), lambda j: (0, 1)),
                  const2(p['d2b']),
                  pl.BlockSpec((512, 512), lambda j: (0, 0)),
                  pl.BlockSpec((512, 512), lambda j: (1, 0)),
                  const2(p['d3b']), const2(p['d4W']), const2(p['d4b']),
                  const2(p['d5W']), const2(p['d5b'])]
    mega_in += tail_in
    in_specs += tail_specs

    out = pl.pallas_call(
        _mega_kernel,
        grid=(_NSTEP,),
        in_specs=in_specs,
        out_specs=pl.BlockSpec((1, 8), lambda j: (0, 0)),
        out_shape=jax.ShapeDtypeStruct((1, 8), f32),
        scratch_shapes=[pltpu.VMEM((24, 128), f32),
                        pltpu.VMEM((16, 128), f32),
                        pltpu.VMEM((1, 2056), f32)],
        compiler_params=pltpu.CompilerParams(
            dimension_semantics=("arbitrary",),
            vmem_limit_bytes=112 * 1024 * 1024,
        ),
    )(*mega_in)
    return out
